# baseline (speedup 1.0000x reference)
import functools

import jax
import jax.numpy as jnp
from jax import lax
from jax.experimental import pallas as pl
from jax.experimental.pallas import tpu as pltpu
from jax.experimental.pallas import tpu_sc as plsc

LATENT = 32
CHUNK = 128
LANES = 16


@functools.lru_cache(maxsize=None)
def _build(B, R):
    info = plsc.get_sparse_core_info()
    nc, ns = info.num_cores, info.num_subcores
    nw = nc * ns
    assert B % (nw * CHUNK) == 0
    b_per_w = B // nw
    n_chunks = b_per_w // CHUNK

    mesh = plsc.VectorSubcoreMesh(core_axis_name="c", subcore_axis_name="s")

    @functools.partial(
        pl.kernel,
        mesh=mesh,
        out_type=jax.ShapeDtypeStruct((B,), jnp.float32),
        compiler_params=pltpu.CompilerParams(needs_layout_passes=False),
        scratch_types=[
            pltpu.VMEM((b_per_w,), jnp.int32),
            pltpu.VMEM((b_per_w,), jnp.int32),
            pltpu.VMEM((LATENT * b_per_w,), jnp.float32),
            pltpu.VMEM((LATENT * b_per_w,), jnp.float32),
            pltpu.VMEM(((LATENT + 1) * LANES,), jnp.float32),
            pltpu.VMEM((b_per_w,), jnp.float32),
            pltpu.SemaphoreType.DMA,
            pltpu.SemaphoreType.DMA,
        ],
    )
    def gmf(user_hbm, item_hbm, ut_hbm, it_hbm, wb_hbm, out_hbm,
            uidx_v, iidx_v, ubuf_v, ibuf_v, wb_v, out_v, sem_u, sem_i):
        wid = lax.axis_index("s") * nc + lax.axis_index("c")
        base = wid * b_per_w

        pltpu.sync_copy(user_hbm.at[pl.ds(base, b_per_w)], uidx_v)
        pltpu.sync_copy(item_hbm.at[pl.ds(base, b_per_w)], iidx_v)
        pltpu.sync_copy(wb_hbm, wb_v)

        def fire(c, carry):
            off = c * R
            cb = c * b_per_w
            for k in range(b_per_w // LANES):
                ksl = pl.ds(k * LANES, LANES)
                dsl = pl.ds(cb + k * LANES, LANES)
                pltpu.async_copy(
                    ut_hbm.at[uidx_v[ksl] + off], ubuf_v.at[dsl], sem_u)
                pltpu.async_copy(
                    it_hbm.at[iidx_v[ksl] + off], ibuf_v.at[dsl], sem_i)
            return carry

        lax.fori_loop(0, LATENT, fire, 0)

        pltpu.make_async_copy(
            ut_hbm.at[pl.ds(0, LATENT * b_per_w)], ubuf_v, sem_u).wait()
        pltpu.make_async_copy(
            it_hbm.at[pl.ds(0, LATENT * b_per_w)], ibuf_v, sem_i).wait()

        bvec = wb_v[pl.ds(LATENT * LANES, LANES)]

        def comp(t, carry):
            sl = pl.ds(t * LANES, LANES)
            acc = bvec
            for c in range(LATENT):
                wc = wb_v[pl.ds(c * LANES, LANES)]
                csl = pl.ds(c * b_per_w + t * LANES, LANES)
                acc = acc + wc * ubuf_v[csl] * ibuf_v[csl]
            out_v[sl] = acc
            return carry

        lax.fori_loop(0, b_per_w // LANES, comp, 0)

        pltpu.sync_copy(out_v, out_hbm.at[pl.ds(base, b_per_w)])

    return gmf


def kernel(user, item, u_table, i_table, W, b):
    B = user.shape[0]
    R = u_table.shape[0]
    user1d = user.reshape(B)
    item1d = item.reshape(B)
    ut_flat = u_table.T.reshape(-1)
    it_flat = i_table.T.reshape(-1)
    wfull = jnp.concatenate([
        jnp.broadcast_to(W.reshape(LATENT, 1), (LATENT, LANES)).reshape(-1),
        jnp.broadcast_to(b.reshape(1), (LANES,)),
    ])
    gmf = _build(B, R)
    out = gmf(user1d, item1d, ut_flat, it_flat, wfull)
    return out.reshape(B, 1)


# speedup vs baseline: 5.7267x; 5.7267x over previous
import functools

import jax
import jax.numpy as jnp
from jax import lax
from jax.experimental import pallas as pl
from jax.experimental.pallas import tpu as pltpu
from jax.experimental.pallas import tpu_sc as plsc

LATENT = 32
CHUNK = 128
LANES = 16


@functools.lru_cache(maxsize=None)
def _build(B):
    info = plsc.get_sparse_core_info()
    nc, ns = info.num_cores, info.num_subcores
    nw = nc * ns
    assert B % (nw * CHUNK) == 0
    b_per_w = B // nw
    n_chunks = b_per_w // CHUNK

    mesh = plsc.VectorSubcoreMesh(core_axis_name="c", subcore_axis_name="s")

    @functools.partial(
        pl.kernel,
        mesh=mesh,
        out_type=jax.ShapeDtypeStruct((B,), jnp.float32),
        compiler_params=pltpu.CompilerParams(
            needs_layout_passes=False, use_tc_tiling_on_sc=False),
        scratch_types=[
            pltpu.VMEM((n_chunks, CHUNK), jnp.int32),
            pltpu.VMEM((n_chunks, CHUNK), jnp.int32),
            pltpu.VMEM((b_per_w, LATENT), jnp.float32),
            pltpu.VMEM((b_per_w, LATENT), jnp.float32),
            pltpu.VMEM((3 * LANES,), jnp.float32),
            pltpu.VMEM((b_per_w,), jnp.float32),
            pltpu.SemaphoreType.DMA,
            pltpu.SemaphoreType.DMA,
        ],
    )
    def gmf(user_hbm, item_hbm, ut_hbm, it_hbm, wb_hbm, out_hbm,
            uidx_v, iidx_v, urows_v, irows_v, wb_v, out_v, sem_u, sem_i):
        wid = lax.axis_index("s") * nc + lax.axis_index("c")
        base = wid * b_per_w
        cbase = wid * n_chunks

        pltpu.sync_copy(user_hbm.at[pl.ds(cbase, n_chunks)], uidx_v)
        pltpu.sync_copy(item_hbm.at[pl.ds(cbase, n_chunks)], iidx_v)

        copies = []
        for c in range(n_chunks):
            dst = pl.ds(c * CHUNK, CHUNK)
            copies.append(pltpu.async_copy(
                ut_hbm.at[uidx_v.at[c]], urows_v.at[dst], sem_u))
            copies.append(pltpu.async_copy(
                it_hbm.at[iidx_v.at[c]], irows_v.at[dst], sem_i))
        pltpu.sync_copy(wb_hbm, wb_v)
        for cp in copies:
            cp.wait()

        w0 = wb_v[pl.ds(0, LANES)]
        w1 = wb_v[pl.ds(LANES, LANES)]
        bfrac = wb_v[pl.ds(2 * LANES, LANES)]
        lane = lax.iota(jnp.int32, LANES)

        def blk(i, carry):
            acc = bfrac * 0.0
            for j in range(LANES):
                r = i * LANES + j
                u0 = urows_v[r, pl.ds(0, LANES)]
                u1 = urows_v[r, pl.ds(LANES, LANES)]
                i0 = irows_v[r, pl.ds(0, LANES)]
                i1 = irows_v[r, pl.ds(LANES, LANES)]
                s = u0 * i0 * w0 + u1 * i1 * w1 + bfrac
                acc = jnp.where(lane == j, jnp.sum(s), acc)
            out_v[pl.ds(i * LANES, LANES)] = acc
            return carry

        lax.fori_loop(0, b_per_w // LANES, blk, 0)

        pltpu.sync_copy(out_v, out_hbm.at[pl.ds(base, b_per_w)])

    return gmf


def kernel(user, item, u_table, i_table, W, b):
    B = user.shape[0]
    user2d = user.reshape(B // CHUNK, CHUNK)
    item2d = item.reshape(B // CHUNK, CHUNK)
    wb = jnp.concatenate(
        [W.reshape(-1), jnp.broadcast_to(b.reshape(-1) / LANES, (LANES,))])
    gmf = _build(B)
    out = gmf(user2d, item2d, u_table, i_table, wb)
    return out.reshape(B, 1)
